# Initial kernel scaffold; baseline (speedup 1.0000x reference)
import jax
import jax.numpy as jnp
from jax.experimental import pallas as pl


def kernel(x, edge_index, W, W_ih, W_hh, b_ih, b_hh):
    raise NotImplementedError("write your pallas kernel here")


# speedup vs baseline: 5.0605x; 5.0605x over previous
import functools

import jax
import jax.numpy as jnp
from jax import lax
from jax.experimental import pallas as pl
from jax.experimental.pallas import tpu as pltpu
from jax.experimental.pallas import tpu_sc as plsc


def _pre_body(x_ref, w_ref, whh_t_ref, bhh_ref, m_ref, gh_ref):
    xb = x_ref[...]
    m_ref[...] = jnp.dot(xb, w_ref[...], preferred_element_type=jnp.float32)
    gh_ref[...] = (
        jnp.dot(xb, whh_t_ref[...], preferred_element_type=jnp.float32)
        + bhh_ref[...]
    )


def _pre_call(x, w, whh_t, bhh, bn):
    n, d = x.shape
    d3 = whh_t.shape[1]
    grid = n // bn
    return pl.pallas_call(
        _pre_body,
        grid=(grid,),
        in_specs=[
            pl.BlockSpec((bn, d), lambda i: (i, 0)),
            pl.BlockSpec((d, d), lambda i: (0, 0)),
            pl.BlockSpec((d, d3), lambda i: (0, 0)),
            pl.BlockSpec((1, d3), lambda i: (0, 0)),
        ],
        out_specs=[
            pl.BlockSpec((bn, d), lambda i: (i, 0)),
            pl.BlockSpec((bn, d3), lambda i: (i, 0)),
        ],
        out_shape=[
            jax.ShapeDtypeStruct((n, d), jnp.float32),
            jax.ShapeDtypeStruct((n, d3), jnp.float32),
        ],
    )(x, w, whh_t, bhh)


def _make_sc_seg_sum(n, d, e):
    info = plsc.get_sparse_core_info()
    nc, ns = info.num_cores, info.num_subcores
    nw = nc * ns
    epw = e // nw
    k = 80
    nchunk = epw // k
    rows_per_tile = (-(-n // ns) + 7) // 8 * 8
    npad = rows_per_tile * ns

    mesh = plsc.VectorSubcoreMesh(core_axis_name="c", subcore_axis_name="s")

    @functools.partial(
        pl.kernel,
        out_type=jax.ShapeDtypeStruct((nc, npad, d), jnp.float32),
        mesh=mesh,
        scratch_types=[
            pltpu.VMEM((k,), jnp.int32),
            pltpu.VMEM((k,), jnp.int32),
            pltpu.VMEM((k, d), jnp.float32),
            pltpu.VMEM_SHARED((npad, d), jnp.float32),
            pltpu.SemaphoreType.DMA,
        ],
    )
    def seg_sum(m_hbm, src_hbm, dst_hbm, zeros_hbm, out_hbm,
                src_v, dst_v, rows_v, agg_sh, sem):
        cid = lax.axis_index("c")
        sid = lax.axis_index("s")
        wid = sid * nc + cid

        row0 = sid * rows_per_tile
        pltpu.sync_copy(zeros_hbm, agg_sh.at[pl.ds(row0, rows_per_tile)])
        plsc.subcore_barrier()

        base0 = wid * epw

        def body(i, carry):
            base = base0 + i * k
            pltpu.sync_copy(src_hbm.at[pl.ds(base, k)], src_v)
            pltpu.sync_copy(dst_hbm.at[pl.ds(base, k)], dst_v)
            pltpu.async_copy(m_hbm.at[src_v], rows_v, sem).wait()
            pltpu.sync_copy(rows_v, agg_sh.at[dst_v], add=True)
            return carry

        lax.fori_loop(0, nchunk, body, 0)
        plsc.subcore_barrier()

        pltpu.sync_copy(
            agg_sh.at[pl.ds(row0, rows_per_tile)],
            out_hbm.at[cid, pl.ds(row0, rows_per_tile)],
        )

    return seg_sum


def _post_body(p0_ref, p1_ref, x_ref, gh_ref, wih_t_ref, bih_ref, out_ref):
    d = x_ref.shape[1]
    agg = p0_ref[...] + p1_ref[...]
    gi = (
        jnp.dot(agg, wih_t_ref[...], preferred_element_type=jnp.float32)
        + bih_ref[...]
    )
    gh = gh_ref[...]
    xb = x_ref[...]
    r = jax.nn.sigmoid(gi[:, :d] + gh[:, :d])
    z = jax.nn.sigmoid(gi[:, d:2 * d] + gh[:, d:2 * d])
    nn = jnp.tanh(gi[:, 2 * d:] + r * gh[:, 2 * d:])
    h = (1.0 - z) * nn + z * xb
    out_ref[...] = xb + jnp.maximum(h, 0.0)


def _post_call(p0, p1, x, gh, wih_t, bih, bn):
    n, d = x.shape
    d3 = wih_t.shape[1]
    grid = n // bn
    return pl.pallas_call(
        _post_body,
        grid=(grid,),
        in_specs=[
            pl.BlockSpec((bn, d), lambda i: (i, 0)),
            pl.BlockSpec((bn, d), lambda i: (i, 0)),
            pl.BlockSpec((bn, d), lambda i: (i, 0)),
            pl.BlockSpec((bn, d3), lambda i: (i, 0)),
            pl.BlockSpec((d, d3), lambda i: (0, 0)),
            pl.BlockSpec((1, d3), lambda i: (0, 0)),
        ],
        out_specs=pl.BlockSpec((bn, d), lambda i: (i, 0)),
        out_shape=jax.ShapeDtypeStruct((n, d), jnp.float32),
    )(p0, p1, x, gh, wih_t, bih)


def kernel(x, edge_index, W, W_ih, W_hh, b_ih, b_hh):
    n, d = x.shape
    e = edge_index.shape[1]
    src = edge_index[0]
    dst = edge_index[1]

    bn = 1000
    m, gh = _pre_call(x, W, W_hh.T, b_hh.reshape(1, -1), bn)

    rows_per_tile = (-(-n // 16) + 7) // 8 * 8
    zeros = jnp.zeros((rows_per_tile, d), jnp.float32)
    part = _make_sc_seg_sum(n, d, e)(m, src, dst, zeros)

    return _post_call(part[0, :n], part[1, :n], x, gh, W_ih.T,
                      b_ih.reshape(1, -1), bn)
